# baseline (speedup 1.0000x reference)
import functools

import jax
import jax.numpy as jnp
from jax import lax
from jax.experimental import pallas as pl
from jax.experimental.pallas import tpu as pltpu
from jax.experimental.pallas import tpu_sc as plsc

_NC, _NS = 2, 16
_NW = _NC * _NS
_CH = 32


def _sc_body(w_hbm, o_hbm, buf, in_sem, out_sem):
    bsz = o_hbm.shape[0]
    seq = w_hbm.shape[0]
    rows_per_w = seq // _NW
    nch = rows_per_w // _CH
    wid = lax.axis_index("s") * _NC + lax.axis_index("c")
    base = wid * rows_per_w

    def in_copy(k, slot):
        return pltpu.make_async_copy(
            w_hbm.at[pl.ds(base + k * _CH, _CH), :],
            buf.at[slot],
            in_sem.at[slot],
        )

    def out_copy(b, k, slot):
        return pltpu.make_async_copy(
            buf.at[slot],
            o_hbm.at[b, pl.ds(base + k * _CH, _CH), :],
            out_sem.at[slot, b],
        )

    for k in range(nch):
        slot = k % 2
        if k >= 2:
            for b in range(bsz):
                out_copy(b, k - 2, slot).wait()
        cp = in_copy(k, slot)
        cp.start()
        cp.wait()
        for b in range(bsz):
            out_copy(b, k, slot).start()
    for k in range(max(nch - 2, 0), nch):
        for b in range(bsz):
            out_copy(b, k, k % 2).wait()


def kernel(input, embedding_weight):
    bsz, seq_len = input.shape
    d = embedding_weight.shape[1]
    run = functools.partial(
        pl.kernel,
        out_type=jax.ShapeDtypeStruct((bsz, seq_len, d), embedding_weight.dtype),
        mesh=plsc.VectorSubcoreMesh(
            core_axis_name="c", subcore_axis_name="s",
            num_cores=_NC, num_subcores=_NS,
        ),
        scratch_types=[
            pltpu.MemorySpace.VMEM((2, _CH, d), embedding_weight.dtype),
            pltpu.SemaphoreType.DMA((2,)),
            pltpu.SemaphoreType.DMA((2, bsz)),
        ],
    )(_sc_body)
    return run(embedding_weight[:seq_len])
